# Initial kernel scaffold; baseline (speedup 1.0000x reference)
import jax
import jax.numpy as jnp
from jax.experimental import pallas as pl


def kernel(price_input, hyp_input_T, hyp_input, gru_Wih, gru_Whh, gru_bih, gru_bhh, att_Win, att_Wout, ae, ab, h1_W, h1_b, h2_W, h2_b, w1, w2, a_p, lin_W, lin_b):
    raise NotImplementedError("write your pallas kernel here")


# speedup vs baseline: 7.4904x; 7.4904x over previous
import jax
import jax.numpy as jnp
from jax import lax
from jax.experimental import pallas as pl
from jax.experimental.pallas import tpu as pltpu
from jax.experimental.pallas import tpu_sc as plsc

_N = 1026
_T = 16
_D = 64
_NNZ = 8192
_NSETS = 17
_CHUNK = 128
_NCHUNK = _NNZ // _CHUNK
_NPAD = 1040


def _tc_gru_att_body(xs_ref, wih_ref, whh_ref, bih_ref, bhh_ref, win_ref,
                     woa_ref, wob_ref, ae_ref, ab_ref, out_ref, ctx_ref):
    wih = wih_ref[...]
    whh = whh_ref[...]
    bih = bih_ref[...]
    bhh = bhh_ref[...]
    h = jnp.zeros((_N, _D), jnp.float32)
    for t in range(_T):
        gi = jnp.dot(xs_ref[t], wih, preferred_element_type=jnp.float32) + bih
        gh = jnp.dot(h, whh, preferred_element_type=jnp.float32) + bhh
        r = jax.nn.sigmoid(gi[:, :_D] + gh[:, :_D])
        z = jax.nn.sigmoid(gi[:, _D:2 * _D] + gh[:, _D:2 * _D])
        n = jnp.tanh(gi[:, 2 * _D:] + r * gh[:, 2 * _D:])
        h = (1.0 - z) * n + z * h
        ctx_ref[t] = h
    q = jnp.dot(h, win_ref[...], preferred_element_type=jnp.float32)
    scores = [jnp.sum(q * ctx_ref[t], axis=1, keepdims=True) for t in range(_T)]
    m = scores[0]
    for t in range(1, _T):
        m = jnp.maximum(m, scores[t])
    es = [jnp.exp(s - m) for s in scores]
    tot = es[0]
    for t in range(1, _T):
        tot = tot + es[t]
    ae = ae_ref[...]
    ab = ab_ref[...]
    mix2 = jnp.zeros((_N, _D), jnp.float32)
    for t in range(_T):
        wt = es[t] / tot
        mt = wt * ctx_ref[t]
        bt = jnp.exp(ab * jnp.float32(-(_T - 1 - t)))
        mix2 = mix2 + jnp.maximum(ae * mt * bt, 0.0) + mt
    out_ref[...] = jnp.tanh(
        jnp.dot(mix2, woa_ref[...], preferred_element_type=jnp.float32)
        + jnp.dot(q, wob_ref[...], preferred_element_type=jnp.float32))


def _tc_mm_relu_body(p_ref, w_ref, b_ref, out_ref):
    v = jnp.dot(p_ref[...], w_ref[...], preferred_element_type=jnp.float32) + b_ref[...]
    out_ref[...] = jnp.where(v > 0, v, 0.2 * v)


def _tc_tail_body(p_ref, w_ref, b_ref, ap_ref, w1_ref, w2_ref, lwa_ref,
                  lwb_ref, lb_ref, out_ref):
    w = w_ref[...]
    b = b_ref[...]
    x2 = []
    for i in range(_NSETS):
        v = jnp.dot(p_ref[i], w, preferred_element_type=jnp.float32) + b
        x2.append(jnp.where(v > 0, v, 0.2 * v))
    sub = [x2[t + 1] - x2[t] for t in range(_T - 1)]
    inv_nd = jnp.float32(1.0 / (_N * _D))
    zs = []
    for t in range(_T - 1):
        s1 = jnp.sum(sub[t], axis=0, keepdims=True)
        bm = jnp.sum(s1, axis=1, keepdims=True) * inv_nd
        apt = ap_ref[t:t + 1, :]
        u = 1.0 / (1.0 + apt * (bm - sub[t]))
        us = (u * sub[t]) / u
        z1 = jnp.sum(us, axis=0, keepdims=True)
        zs.append(jnp.sum(z1, axis=1, keepdims=True))
    v = jnp.zeros((_D, 1), jnp.float32)
    w1 = w1_ref[...]
    for t in range(_T - 1):
        v = v + w1[:, t:t + 1] * zs[t]
    a = jnp.where(v > 0, v, 0.2 * v)
    s2 = jnp.dot(w2_ref[...], a, preferred_element_type=jnp.float32)
    m = jnp.max(s2, axis=0, keepdims=True)
    e = jnp.exp(s2 - m)
    wat = e / jnp.sum(e, axis=0, keepdims=True)
    xx1 = wat[0:1, 0:1] * sub[0] + wat[2:3, 0:1] * sub[2]
    res = (jnp.dot(x2[_NSETS - 1], lwa_ref[...], preferred_element_type=jnp.float32)
           + jnp.dot(xx1, lwb_ref[...], preferred_element_type=jnp.float32)
           + lb_ref[...])
    out_ref[...] = jnp.where(res > 0, res, 0.01 * res)


def _sc_seg_body(x_hbm, idx_hbm, out_hbm, idx_row, idx_col, dinv, binv, gbuf,
                 sbuf, zch, acc_sh, sem):
    sid = lax.axis_index("s")
    wid = sid * 2 + lax.axis_index("c")
    nsrc = x_hbm.shape[0]
    my_acc = acc_sh.at[sid]

    @pl.when(wid < _NSETS)
    def _():
        i = wid
        isrc = lax.rem(i, nsrc)
        zeros16 = jnp.zeros((16,), jnp.float32)
        ones16 = jnp.ones((16,), jnp.float32)

        pltpu.sync_copy(idx_hbm.at[i, 0], idx_row)
        pltpu.sync_copy(idx_hbm.at[i, 1], idx_col)

        def zdeg(k, c):
            dinv[pl.ds(k * 16, 16)] = zeros16
            binv[pl.ds(k * 16, 16)] = zeros16
            return c

        lax.fori_loop(0, _NPAD // 16, zdeg, 0)

        def cnt(j, c):
            a = j // 8
            s = lax.rem(j, 8) * 16
            plsc.addupdate_scatter(dinv, [idx_row[a, pl.ds(s, 16)]], ones16)
            plsc.addupdate_scatter(binv, [idx_col[a, pl.ds(s, 16)]], ones16)
            return c

        lax.fori_loop(0, _NNZ // 16, cnt, 0)

        def rcp(k, c):
            sl = pl.ds(k * 16, 16)
            dv = dinv[sl]
            dinv[sl] = jnp.where(dv > 0, 1.0 / dv, 0.0)
            bv = binv[sl]
            binv[sl] = jnp.where(bv > 0, 1.0 / bv, 0.0)
            return c

        lax.fori_loop(0, _NPAD // 16, rcp, 0)

        def zz(r, c):
            for f in range(_D // 16):
                zch[r, pl.ds(f * 16, 16)] = zeros16
            return c

        lax.fori_loop(0, _CHUNK, zz, 0)

        def zero_acc():
            def zb(c, x):
                pltpu.sync_copy(zch, my_acc.at[pl.ds(c * _CHUNK, _CHUNK)])
                return x

            lax.fori_loop(0, 8, zb, 0)
            pltpu.sync_copy(zch.at[pl.ds(0, 16)], my_acc.at[pl.ds(1024, 16)])

        lane = lax.iota(jnp.int32, 16)

        def scale_rows(sref, base, ngroups):
            def gw(k, y):
                rows = k * 16 + lane
                sv = sref[pl.ds(base + k * 16, 16)]
                for f in range(_D):
                    fv = jnp.full((16,), f, jnp.int32)
                    g = plsc.load_gather(sbuf, [rows, fv])
                    plsc.store_scatter(sbuf, [rows, fv], g * sv)
                return y

            lax.fori_loop(0, ngroups, gw, 0)

        def scale_out(sref):
            def ch(c, x):
                pltpu.sync_copy(my_acc.at[pl.ds(c * _CHUNK, _CHUNK)], sbuf)
                scale_rows(sref, c * _CHUNK, _CHUNK // 16)
                pltpu.sync_copy(sbuf, out_hbm.at[i].at[pl.ds(c * _CHUNK, _CHUNK)])
                return x

            lax.fori_loop(0, 8, ch, 0)
            pltpu.sync_copy(my_acc.at[pl.ds(1024, 16)], sbuf.at[pl.ds(0, 16)])
            scale_rows(sref, 1024, 1)
            pltpu.sync_copy(sbuf.at[pl.ds(0, 2)], out_hbm.at[i].at[pl.ds(1024, 2)])

        zero_acc()

        def epass(c, carry):
            pltpu.async_copy(x_hbm.at[isrc].at[idx_row.at[c]], gbuf, sem).wait()
            pltpu.sync_copy(gbuf, my_acc.at[idx_col.at[c]], add=True)
            return carry

        lax.fori_loop(0, _NCHUNK, epass, 0)
        scale_out(binv)
        zero_acc()

        def npass(c, carry):
            pltpu.async_copy(out_hbm.at[i].at[idx_col.at[c]], gbuf, sem).wait()
            pltpu.sync_copy(gbuf, my_acc.at[idx_row.at[c]], add=True)
            return carry

        lax.fori_loop(0, _NCHUNK, npass, 0)
        scale_out(dinv)


def _sc_seg(x_tables, idx_all):
    mesh = plsc.VectorSubcoreMesh(core_axis_name="c", subcore_axis_name="s",
                                  num_cores=2, num_subcores=16)
    return pl.kernel(
        _sc_seg_body,
        out_type=jax.ShapeDtypeStruct((_NSETS, _N, _D), jnp.float32),
        mesh=mesh,
        compiler_params=pltpu.CompilerParams(needs_layout_passes=False,
                                             use_tc_tiling_on_sc=False),
        scratch_types=[
            pltpu.VMEM((_NCHUNK, _CHUNK), jnp.int32),
            pltpu.VMEM((_NCHUNK, _CHUNK), jnp.int32),
            pltpu.VMEM((_NPAD,), jnp.float32),
            pltpu.VMEM((_NPAD,), jnp.float32),
            pltpu.VMEM((_CHUNK, _D), jnp.float32),
            pltpu.VMEM((_CHUNK, _D), jnp.float32),
            pltpu.VMEM((_CHUNK, _D), jnp.float32),
            pltpu.VMEM_SHARED((16, _NPAD, _D), jnp.float32),
            pltpu.SemaphoreType.DMA,
        ],
    )(x_tables, idx_all)


def kernel(price_input, hyp_input_T, hyp_input, gru_Wih, gru_Whh, gru_bih,
           gru_bhh, att_Win, att_Wout, ae, ab, h1_W, h1_b, h2_W, h2_b, w1,
           w2, a_p, lin_W, lin_b):
    f32 = jnp.float32
    xs = jnp.transpose(price_input, (1, 0, 2))
    idx_all = jnp.concatenate([hyp_input_T, hyp_input[None]], axis=0)
    idx_all = idx_all.reshape(_NSETS, 2, _NCHUNK, _CHUNK)

    output = pl.pallas_call(
        _tc_gru_att_body,
        out_shape=jax.ShapeDtypeStruct((_N, _D), f32),
        scratch_shapes=[pltpu.VMEM((_T, _N, _D), f32)],
    )(xs, gru_Wih.T, gru_Whh.T, gru_bih.reshape(1, 3 * _D),
      gru_bhh.reshape(1, 3 * _D), att_Win.T, att_Wout[:, :_D].T,
      att_Wout[:, _D:].T, ae.reshape(_N, 1), ab.reshape(_N, 1))

    p1 = _sc_seg(output[None], idx_all)

    x1 = pl.pallas_call(
        _tc_mm_relu_body,
        out_shape=jax.ShapeDtypeStruct((_NSETS * _N, _D), f32),
    )(p1.reshape(_NSETS * _N, _D), h1_W.T, h1_b.reshape(1, _D))

    p2 = _sc_seg(x1.reshape(_NSETS, _N, _D), idx_all)

    out = pl.pallas_call(
        _tc_tail_body,
        out_shape=jax.ShapeDtypeStruct((_N, 1), f32),
    )(p2, h2_W.T, h2_b.reshape(1, _D), a_p.reshape(_T - 1, 1), w1, w2,
      lin_W[:, :_D].T, lin_W[:, _D:].T, lin_b.reshape(1, 1))
    return out


# speedup vs baseline: 10.2031x; 1.3621x over previous
import jax
import jax.numpy as jnp
from jax import lax
from jax.experimental import pallas as pl
from jax.experimental.pallas import tpu as pltpu
from jax.experimental.pallas import tpu_sc as plsc

_N = 1026
_T = 16
_D = 64
_NNZ = 8192
_NSETS = 17
_CHUNK = 128
_NCHUNK = _NNZ // _CHUNK
_NPAD = 1040


def _tc_gru_att_body(xs_ref, wih_ref, whh_ref, bih_ref, bhh_ref, win_ref,
                     woa_ref, wob_ref, ae_ref, ab_ref, out_ref, ctx_ref):
    wih = wih_ref[...]
    whh = whh_ref[...]
    bih = bih_ref[...]
    bhh = bhh_ref[...]
    h = jnp.zeros((_N, _D), jnp.float32)
    for t in range(_T):
        gi = jnp.dot(xs_ref[t], wih, preferred_element_type=jnp.float32) + bih
        gh = jnp.dot(h, whh, preferred_element_type=jnp.float32) + bhh
        r = jax.nn.sigmoid(gi[:, :_D] + gh[:, :_D])
        z = jax.nn.sigmoid(gi[:, _D:2 * _D] + gh[:, _D:2 * _D])
        n = jnp.tanh(gi[:, 2 * _D:] + r * gh[:, 2 * _D:])
        h = (1.0 - z) * n + z * h
        ctx_ref[t] = h
    q = jnp.dot(h, win_ref[...], preferred_element_type=jnp.float32)
    scores = [jnp.sum(q * ctx_ref[t], axis=1, keepdims=True) for t in range(_T)]
    m = scores[0]
    for t in range(1, _T):
        m = jnp.maximum(m, scores[t])
    es = [jnp.exp(s - m) for s in scores]
    tot = es[0]
    for t in range(1, _T):
        tot = tot + es[t]
    ae = ae_ref[...]
    ab = ab_ref[...]
    mix2 = jnp.zeros((_N, _D), jnp.float32)
    for t in range(_T):
        wt = es[t] / tot
        mt = wt * ctx_ref[t]
        bt = jnp.exp(ab * jnp.float32(-(_T - 1 - t)))
        mix2 = mix2 + jnp.maximum(ae * mt * bt, 0.0) + mt
    out_ref[...] = jnp.tanh(
        jnp.dot(mix2, woa_ref[...], preferred_element_type=jnp.float32)
        + jnp.dot(q, wob_ref[...], preferred_element_type=jnp.float32))


def _tc_mm_relu_body(p_ref, w_ref, b_ref, out_ref):
    v = jnp.dot(p_ref[...], w_ref[...], preferred_element_type=jnp.float32) + b_ref[...]
    out_ref[...] = jnp.where(v > 0, v, 0.2 * v)


def _tc_tail_body(p_ref, w_ref, b_ref, ap_ref, w1_ref, w2_ref, lwa_ref,
                  lwb_ref, lb_ref, out_ref):
    w = w_ref[...]
    b = b_ref[...]
    x2 = []
    for i in range(_NSETS):
        v = jnp.dot(p_ref[i], w, preferred_element_type=jnp.float32) + b
        x2.append(jnp.where(v > 0, v, 0.2 * v))
    sub = [x2[t + 1] - x2[t] for t in range(_T - 1)]
    inv_nd = jnp.float32(1.0 / (_N * _D))
    zs = []
    for t in range(_T - 1):
        s1 = jnp.sum(sub[t], axis=0, keepdims=True)
        bm = jnp.sum(s1, axis=1, keepdims=True) * inv_nd
        apt = ap_ref[t:t + 1, :]
        u = 1.0 / (1.0 + apt * (bm - sub[t]))
        us = (u * sub[t]) / u
        z1 = jnp.sum(us, axis=0, keepdims=True)
        zs.append(jnp.sum(z1, axis=1, keepdims=True))
    v = jnp.zeros((_D, 1), jnp.float32)
    w1 = w1_ref[...]
    for t in range(_T - 1):
        v = v + w1[:, t:t + 1] * zs[t]
    a = jnp.where(v > 0, v, 0.2 * v)
    s2 = jnp.dot(w2_ref[...], a, preferred_element_type=jnp.float32)
    m = jnp.max(s2, axis=0, keepdims=True)
    e = jnp.exp(s2 - m)
    wat = e / jnp.sum(e, axis=0, keepdims=True)
    xx1 = wat[0:1, 0:1] * sub[0] + wat[2:3, 0:1] * sub[2]
    res = (jnp.dot(x2[_NSETS - 1], lwa_ref[...], preferred_element_type=jnp.float32)
           + jnp.dot(xx1, lwb_ref[...], preferred_element_type=jnp.float32)
           + lb_ref[...])
    out_ref[...] = jnp.where(res > 0, res, 0.01 * res)


def _sc_seg_body(x_hbm, idx_hbm, out_hbm, idx_row, idx_col, dinv, binv, gbufs,
                 sbuf, zch, acc_sh, gsem, ssem):
    sid = lax.axis_index("s")
    wid = sid * 2 + lax.axis_index("c")
    nsrc = x_hbm.shape[0]
    my_acc = acc_sh.at[sid]

    @pl.when(wid < _NSETS)
    def _():
        i = wid
        isrc = lax.rem(i, nsrc)
        zeros16 = jnp.zeros((16,), jnp.float32)
        ones16 = jnp.ones((16,), jnp.float32)

        pltpu.sync_copy(idx_hbm.at[i, 0], idx_row)
        pltpu.sync_copy(idx_hbm.at[i, 1], idx_col)

        gds = [None] * _NCHUNK

        def prime(tbl, gidx):
            for b in range(2):
                gds[b] = pltpu.async_copy(tbl.at[gidx.at[b]], gbufs.at[b], gsem)

        def finish(tbl, gidx, sidx):
            sds = [None] * _NCHUNK
            ngroups = _NCHUNK // 2
            for o in range(ngroups):
                bank = (o % 2) * 2
                nbank = ((o + 1) % 2) * 2
                if o > 0:
                    for b in range(2):
                        sds[(o - 1) * 2 + b].wait()
                if o < ngroups - 1:
                    for b in range(2):
                        c = (o + 1) * 2 + b
                        gds[c] = pltpu.async_copy(tbl.at[gidx.at[c]],
                                                  gbufs.at[nbank + b], gsem)
                for b in range(2):
                    c = o * 2 + b
                    gds[c].wait()
                    sds[c] = pltpu.async_copy(gbufs.at[bank + b],
                                              my_acc.at[sidx.at[c]], ssem,
                                              add=True)
            for b in range(2):
                sds[_NCHUNK - 2 + b].wait()

        prime(x_hbm.at[isrc], idx_row)

        def zdeg(k, c):
            dinv[pl.ds(k * 16, 16)] = zeros16
            binv[pl.ds(k * 16, 16)] = zeros16
            return c

        lax.fori_loop(0, _NPAD // 16, zdeg, 0)

        def cnt(j, c):
            a = j // 8
            s = lax.rem(j, 8) * 16
            plsc.addupdate_scatter(dinv, [idx_row[a, pl.ds(s, 16)]], ones16)
            plsc.addupdate_scatter(binv, [idx_col[a, pl.ds(s, 16)]], ones16)
            return c

        lax.fori_loop(0, _NNZ // 16, cnt, 0)

        def rcp(k, c):
            sl = pl.ds(k * 16, 16)
            dv = dinv[sl]
            dinv[sl] = jnp.where(dv > 0, 1.0 / dv, 0.0)
            bv = binv[sl]
            binv[sl] = jnp.where(bv > 0, 1.0 / bv, 0.0)
            return c

        lax.fori_loop(0, _NPAD // 16, rcp, 0)

        def zz(r, c):
            for f in range(_D // 16):
                zch[r, pl.ds(f * 16, 16)] = zeros16
            return c

        lax.fori_loop(0, _CHUNK, zz, 0)

        def zero_acc():
            def zb(c, x):
                pltpu.sync_copy(zch, my_acc.at[pl.ds(c * _CHUNK, _CHUNK)])
                return x

            lax.fori_loop(0, 8, zb, 0)
            pltpu.sync_copy(zch.at[pl.ds(0, 16)], my_acc.at[pl.ds(1024, 16)])

        lane = lax.iota(jnp.int32, 16)

        def scale_rows(sref, base, ngroups):
            def gw(k, y):
                rows = k * 16 + lane
                sv = sref[pl.ds(base + k * 16, 16)]
                for f in range(_D):
                    fv = jnp.full((16,), f, jnp.int32)
                    g = plsc.load_gather(sbuf, [rows, fv])
                    plsc.store_scatter(sbuf, [rows, fv], g * sv)
                return y

            lax.fori_loop(0, ngroups, gw, 0)

        def scale_out(sref):
            def ch(c, x):
                pltpu.sync_copy(my_acc.at[pl.ds(c * _CHUNK, _CHUNK)], sbuf)
                scale_rows(sref, c * _CHUNK, _CHUNK // 16)
                pltpu.sync_copy(sbuf, out_hbm.at[i].at[pl.ds(c * _CHUNK, _CHUNK)])
                return x

            lax.fori_loop(0, 8, ch, 0)
            pltpu.sync_copy(my_acc.at[pl.ds(1024, 16)], sbuf.at[pl.ds(0, 16)])
            scale_rows(sref, 1024, 1)
            pltpu.sync_copy(sbuf.at[pl.ds(0, 2)], out_hbm.at[i].at[pl.ds(1024, 2)])

        zero_acc()
        finish(x_hbm.at[isrc], idx_row, idx_col)
        scale_out(binv)
        zero_acc()
        prime(out_hbm.at[i], idx_col)
        finish(out_hbm.at[i], idx_col, idx_row)
        scale_out(dinv)


def _sc_seg(x_tables, idx_all):
    mesh = plsc.VectorSubcoreMesh(core_axis_name="c", subcore_axis_name="s",
                                  num_cores=2, num_subcores=16)
    return pl.kernel(
        _sc_seg_body,
        out_type=jax.ShapeDtypeStruct((_NSETS, _N, _D), jnp.float32),
        mesh=mesh,
        compiler_params=pltpu.CompilerParams(needs_layout_passes=False,
                                             use_tc_tiling_on_sc=False),
        scratch_types=[
            pltpu.VMEM((_NCHUNK, _CHUNK), jnp.int32),
            pltpu.VMEM((_NCHUNK, _CHUNK), jnp.int32),
            pltpu.VMEM((_NPAD,), jnp.float32),
            pltpu.VMEM((_NPAD,), jnp.float32),
            pltpu.VMEM((4, _CHUNK, _D), jnp.float32),
            pltpu.VMEM((_CHUNK, _D), jnp.float32),
            pltpu.VMEM((_CHUNK, _D), jnp.float32),
            pltpu.VMEM_SHARED((9, _NPAD, _D), jnp.float32),
            pltpu.SemaphoreType.DMA,
            pltpu.SemaphoreType.DMA,
        ],
    )(x_tables, idx_all)


def kernel(price_input, hyp_input_T, hyp_input, gru_Wih, gru_Whh, gru_bih,
           gru_bhh, att_Win, att_Wout, ae, ab, h1_W, h1_b, h2_W, h2_b, w1,
           w2, a_p, lin_W, lin_b):
    f32 = jnp.float32
    xs = jnp.transpose(price_input, (1, 0, 2))
    idx_all = jnp.concatenate([hyp_input_T, hyp_input[None]], axis=0)
    idx_all = idx_all.reshape(_NSETS, 2, _NCHUNK, _CHUNK)

    output = pl.pallas_call(
        _tc_gru_att_body,
        out_shape=jax.ShapeDtypeStruct((_N, _D), f32),
        scratch_shapes=[pltpu.VMEM((_T, _N, _D), f32)],
    )(xs, gru_Wih.T, gru_Whh.T, gru_bih.reshape(1, 3 * _D),
      gru_bhh.reshape(1, 3 * _D), att_Win.T, att_Wout[:, :_D].T,
      att_Wout[:, _D:].T, ae.reshape(_N, 1), ab.reshape(_N, 1))

    p1 = _sc_seg(output[None], idx_all)

    x1 = pl.pallas_call(
        _tc_mm_relu_body,
        out_shape=jax.ShapeDtypeStruct((_NSETS * _N, _D), f32),
    )(p1.reshape(_NSETS * _N, _D), h1_W.T, h1_b.reshape(1, _D))

    p2 = _sc_seg(x1.reshape(_NSETS, _N, _D), idx_all)

    out = pl.pallas_call(
        _tc_tail_body,
        out_shape=jax.ShapeDtypeStruct((_N, 1), f32),
    )(p2, h2_W.T, h2_b.reshape(1, _D), a_p.reshape(_T - 1, 1), w1, w2,
      lin_W[:, :_D].T, lin_W[:, _D:].T, lin_b.reshape(1, 1))
    return out


# speedup vs baseline: 10.4704x; 1.0262x over previous
import jax
import jax.numpy as jnp
from jax import lax
from jax.experimental import pallas as pl
from jax.experimental.pallas import tpu as pltpu
from jax.experimental.pallas import tpu_sc as plsc

_N = 1026
_T = 16
_D = 64
_NNZ = 8192
_NSETS = 17
_CHUNK = 128
_NCHUNK = _NNZ // _CHUNK
_NPAD = 1040


def _tc_gru_att_body(xs_ref, wih_ref, whh_ref, bih_ref, bhh_ref, win_ref,
                     woa_ref, wob_ref, ae_ref, ab_ref, out_ref, ctx_ref):
    wih = wih_ref[...]
    whh = whh_ref[...]
    bih = bih_ref[...]
    bhh = bhh_ref[...]
    h = jnp.zeros((_N, _D), jnp.float32)
    for t in range(_T):
        gi = jnp.dot(xs_ref[t], wih, preferred_element_type=jnp.float32) + bih
        gh = jnp.dot(h, whh, preferred_element_type=jnp.float32) + bhh
        r = jax.nn.sigmoid(gi[:, :_D] + gh[:, :_D])
        z = jax.nn.sigmoid(gi[:, _D:2 * _D] + gh[:, _D:2 * _D])
        n = jnp.tanh(gi[:, 2 * _D:] + r * gh[:, 2 * _D:])
        h = (1.0 - z) * n + z * h
        ctx_ref[t] = h
    q = jnp.dot(h, win_ref[...], preferred_element_type=jnp.float32)
    scores = [jnp.sum(q * ctx_ref[t], axis=1, keepdims=True) for t in range(_T)]
    m = scores[0]
    for t in range(1, _T):
        m = jnp.maximum(m, scores[t])
    es = [jnp.exp(s - m) for s in scores]
    tot = es[0]
    for t in range(1, _T):
        tot = tot + es[t]
    ae = ae_ref[...]
    ab = ab_ref[...]
    mix2 = jnp.zeros((_N, _D), jnp.float32)
    for t in range(_T):
        wt = es[t] / tot
        mt = wt * ctx_ref[t]
        bt = jnp.exp(ab * jnp.float32(-(_T - 1 - t)))
        mix2 = mix2 + jnp.maximum(ae * mt * bt, 0.0) + mt
    out_ref[...] = jnp.tanh(
        jnp.dot(mix2, woa_ref[...], preferred_element_type=jnp.float32)
        + jnp.dot(q, wob_ref[...], preferred_element_type=jnp.float32))


def _tc_mm_relu_body(p_ref, w_ref, b_ref, out_ref):
    v = jnp.dot(p_ref[...], w_ref[...], preferred_element_type=jnp.float32) + b_ref[...]
    out_ref[...] = jnp.where(v > 0, v, 0.2 * v)


def _tc_tail_body(p_ref, w_ref, b_ref, ap_ref, w1_ref, w2_ref, lwa_ref,
                  lwb_ref, lb_ref, out_ref):
    w = w_ref[...]
    b = b_ref[...]
    x2 = []
    for i in range(_NSETS):
        v = jnp.dot(p_ref[i], w, preferred_element_type=jnp.float32) + b
        x2.append(jnp.where(v > 0, v, 0.2 * v))
    sub = [x2[t + 1] - x2[t] for t in range(_T - 1)]
    inv_nd = jnp.float32(1.0 / (_N * _D))
    zs = []
    for t in range(_T - 1):
        s1 = jnp.sum(sub[t], axis=0, keepdims=True)
        bm = jnp.sum(s1, axis=1, keepdims=True) * inv_nd
        apt = ap_ref[t:t + 1, :]
        u = 1.0 / (1.0 + apt * (bm - sub[t]))
        us = (u * sub[t]) / u
        z1 = jnp.sum(us, axis=0, keepdims=True)
        zs.append(jnp.sum(z1, axis=1, keepdims=True))
    v = jnp.zeros((_D, 1), jnp.float32)
    w1 = w1_ref[...]
    for t in range(_T - 1):
        v = v + w1[:, t:t + 1] * zs[t]
    a = jnp.where(v > 0, v, 0.2 * v)
    s2 = jnp.dot(w2_ref[...], a, preferred_element_type=jnp.float32)
    m = jnp.max(s2, axis=0, keepdims=True)
    e = jnp.exp(s2 - m)
    wat = e / jnp.sum(e, axis=0, keepdims=True)
    xx1 = wat[0:1, 0:1] * sub[0] + wat[2:3, 0:1] * sub[2]
    res = (jnp.dot(x2[_NSETS - 1], lwa_ref[...], preferred_element_type=jnp.float32)
           + jnp.dot(xx1, lwb_ref[...], preferred_element_type=jnp.float32)
           + lb_ref[...])
    out_ref[...] = jnp.where(res > 0, res, 0.01 * res)


def _sc_seg_body(x_hbm, idx_hbm, out_hbm, idx_row, idx_col, dinv, binv, gbufs,
                 sbuf, acc_sh, gsem, ssem):
    sid = lax.axis_index("s")
    wid = sid * 2 + lax.axis_index("c")
    nsrc = x_hbm.shape[0]
    my_acc = acc_sh.at[sid]

    @pl.when(wid < _NSETS)
    def _():
        i = wid
        isrc = lax.rem(i, nsrc)
        zeros16 = jnp.zeros((16,), jnp.float32)
        ones16 = jnp.ones((16,), jnp.float32)

        pltpu.sync_copy(idx_hbm.at[i, 0], idx_row)
        pltpu.sync_copy(idx_hbm.at[i, 1], idx_col)

        gds = [None] * _NCHUNK

        def prime(tbl, gidx):
            for b in range(4):
                gds[b] = pltpu.async_copy(tbl.at[gidx.at[b]], gbufs.at[b], gsem)

        def finish(tbl, gidx, sidx):
            sds = [None] * _NCHUNK
            ngroups = _NCHUNK // 4
            for o in range(ngroups):
                bank = (o % 2) * 4
                nbank = ((o + 1) % 2) * 4
                if o > 0:
                    for b in range(4):
                        sds[(o - 1) * 4 + b].wait()
                if o < ngroups - 1:
                    for b in range(4):
                        c = (o + 1) * 4 + b
                        gds[c] = pltpu.async_copy(tbl.at[gidx.at[c]],
                                                  gbufs.at[nbank + b], gsem)
                for b in range(4):
                    c = o * 4 + b
                    gds[c].wait()
                    sds[c] = pltpu.async_copy(gbufs.at[bank + b],
                                              my_acc.at[sidx.at[c]], ssem,
                                              add=True)
            for b in range(4):
                sds[_NCHUNK - 4 + b].wait()

        prime(x_hbm.at[isrc], idx_row)

        def zdeg(k, c):
            dinv[pl.ds(k * 16, 16)] = zeros16
            binv[pl.ds(k * 16, 16)] = zeros16
            return c

        lax.fori_loop(0, _NPAD // 16, zdeg, 0)

        def cnt(j, c):
            a = j // 8
            s = lax.rem(j, 8) * 16
            plsc.addupdate_scatter(dinv, [idx_row[a, pl.ds(s, 16)]], ones16)
            plsc.addupdate_scatter(binv, [idx_col[a, pl.ds(s, 16)]], ones16)
            return c

        lax.fori_loop(0, _NNZ // 16, cnt, 0)

        def rcp(k, c):
            sl = pl.ds(k * 16, 16)
            dv = dinv[sl]
            dinv[sl] = jnp.where(dv > 0, 1.0 / dv, 0.0)
            bv = binv[sl]
            binv[sl] = jnp.where(bv > 0, 1.0 / bv, 0.0)
            return c

        lax.fori_loop(0, _NPAD // 16, rcp, 0)

        def zs(r, c):
            for f in range(_D // 16):
                sbuf[r, pl.ds(f * 16, 16)] = zeros16
            return c

        lax.fori_loop(0, _CHUNK, zs, 0)

        def zero_acc():
            zds = []
            for c in range(8):
                zds.append(pltpu.async_copy(
                    sbuf, my_acc.at[pl.ds(c * _CHUNK, _CHUNK)], ssem))
            zds.append(pltpu.async_copy(
                sbuf.at[pl.ds(0, 16)], my_acc.at[pl.ds(1024, 16)], ssem))
            for d in zds:
                d.wait()

        lane = lax.iota(jnp.int32, 16)

        def scale_rows(bref, sref, base, ngroups):
            def gw(k, y):
                rows = k * 16 + lane
                sv = sref[pl.ds(base + k * 16, 16)]
                for f in range(_D):
                    fv = jnp.full((16,), f, jnp.int32)
                    g = plsc.load_gather(bref, [rows, fv])
                    plsc.store_scatter(bref, [rows, fv], g * sv)
                return y

            lax.fori_loop(0, ngroups, gw, 0)

        def scale_out(sref):
            rds = [None] * 9
            wds = [None] * 9

            def issue_read(c):
                if c < 8:
                    rds[c] = pltpu.async_copy(
                        my_acc.at[pl.ds(c * _CHUNK, _CHUNK)], gbufs.at[c % 4],
                        gsem)
                else:
                    rds[c] = pltpu.async_copy(
                        my_acc.at[pl.ds(1024, 16)],
                        gbufs.at[c % 4].at[pl.ds(0, 16)], gsem)

            def issue_write(c):
                if c < 8:
                    wds[c] = pltpu.async_copy(
                        gbufs.at[c % 4], out_hbm.at[i].at[pl.ds(c * _CHUNK,
                                                               _CHUNK)], ssem)
                else:
                    wds[c] = pltpu.async_copy(
                        gbufs.at[c % 4].at[pl.ds(0, 2)],
                        out_hbm.at[i].at[pl.ds(1024, 2)], ssem)

            issue_read(0)
            issue_read(1)
            for c in range(9):
                if c >= 2:
                    wds[c - 2].wait()
                if c + 2 <= 8:
                    issue_read(c + 2)
                rds[c].wait()
                scale_rows(gbufs.at[c % 4], sref, c * _CHUNK,
                           8 if c < 8 else 1)
                issue_write(c)
            wds[7].wait()
            wds[8].wait()

        zero_acc()
        finish(x_hbm.at[isrc], idx_row, idx_col)
        scale_out(binv)
        zero_acc()
        prime(out_hbm.at[i], idx_col)
        finish(out_hbm.at[i], idx_col, idx_row)
        scale_out(dinv)


def _sc_seg(x_tables, idx_all):
    mesh = plsc.VectorSubcoreMesh(core_axis_name="c", subcore_axis_name="s",
                                  num_cores=2, num_subcores=16)
    return pl.kernel(
        _sc_seg_body,
        out_type=jax.ShapeDtypeStruct((_NSETS, _N, _D), jnp.float32),
        mesh=mesh,
        compiler_params=pltpu.CompilerParams(needs_layout_passes=False,
                                             use_tc_tiling_on_sc=False),
        scratch_types=[
            pltpu.VMEM((_NCHUNK, _CHUNK), jnp.int32),
            pltpu.VMEM((_NCHUNK, _CHUNK), jnp.int32),
            pltpu.VMEM((_NPAD,), jnp.float32),
            pltpu.VMEM((_NPAD,), jnp.float32),
            pltpu.VMEM((8, _CHUNK, _D), jnp.float32),
            pltpu.VMEM((_CHUNK, _D), jnp.float32),
            pltpu.VMEM_SHARED((9, _NPAD, _D), jnp.float32),
            pltpu.SemaphoreType.DMA,
            pltpu.SemaphoreType.DMA,
        ],
    )(x_tables, idx_all)


def kernel(price_input, hyp_input_T, hyp_input, gru_Wih, gru_Whh, gru_bih,
           gru_bhh, att_Win, att_Wout, ae, ab, h1_W, h1_b, h2_W, h2_b, w1,
           w2, a_p, lin_W, lin_b):
    f32 = jnp.float32
    xs = jnp.transpose(price_input, (1, 0, 2))
    idx_all = jnp.concatenate([hyp_input_T, hyp_input[None]], axis=0)
    idx_all = idx_all.reshape(_NSETS, 2, _NCHUNK, _CHUNK)

    output = pl.pallas_call(
        _tc_gru_att_body,
        out_shape=jax.ShapeDtypeStruct((_N, _D), f32),
        scratch_shapes=[pltpu.VMEM((_T, _N, _D), f32)],
    )(xs, gru_Wih.T, gru_Whh.T, gru_bih.reshape(1, 3 * _D),
      gru_bhh.reshape(1, 3 * _D), att_Win.T, att_Wout[:, :_D].T,
      att_Wout[:, _D:].T, ae.reshape(_N, 1), ab.reshape(_N, 1))

    p1 = _sc_seg(output[None], idx_all)

    x1 = pl.pallas_call(
        _tc_mm_relu_body,
        out_shape=jax.ShapeDtypeStruct((_NSETS * _N, _D), f32),
    )(p1.reshape(_NSETS * _N, _D), h1_W.T, h1_b.reshape(1, _D))

    p2 = _sc_seg(x1.reshape(_NSETS, _N, _D), idx_all)

    out = pl.pallas_call(
        _tc_tail_body,
        out_shape=jax.ShapeDtypeStruct((_N, 1), f32),
    )(p2, h2_W.T, h2_b.reshape(1, _D), a_p.reshape(_T - 1, 1), w1, w2,
      lin_W[:, :_D].T, lin_W[:, _D:].T, lin_b.reshape(1, 1))
    return out


# speedup vs baseline: 14.2936x; 1.3651x over previous
import jax
import jax.numpy as jnp
from jax import lax
from jax.experimental import pallas as pl
from jax.experimental.pallas import tpu as pltpu
from jax.experimental.pallas import tpu_sc as plsc

_N = 1026
_T = 16
_D = 64
_NNZ = 8192
_NSETS = 17
_CHUNK = 128
_NCHUNK = _NNZ // _CHUNK
_NPAD = 1040


def _tc_gru_att_body(xs_ref, wih_ref, whh_ref, bih_ref, bhh_ref, win_ref,
                     woa_ref, wob_ref, ae_ref, ab_ref, out_ref, ctx_ref):
    wih = wih_ref[...]
    whh = whh_ref[...]
    bih = bih_ref[...]
    bhh = bhh_ref[...]
    h = jnp.zeros((_N, _D), jnp.float32)
    for t in range(_T):
        gi = jnp.dot(xs_ref[t], wih, preferred_element_type=jnp.float32) + bih
        gh = jnp.dot(h, whh, preferred_element_type=jnp.float32) + bhh
        r = jax.nn.sigmoid(gi[:, :_D] + gh[:, :_D])
        z = jax.nn.sigmoid(gi[:, _D:2 * _D] + gh[:, _D:2 * _D])
        n = jnp.tanh(gi[:, 2 * _D:] + r * gh[:, 2 * _D:])
        h = (1.0 - z) * n + z * h
        ctx_ref[t] = h
    q = jnp.dot(h, win_ref[...], preferred_element_type=jnp.float32)
    scores = [jnp.sum(q * ctx_ref[t], axis=1, keepdims=True) for t in range(_T)]
    m = scores[0]
    for t in range(1, _T):
        m = jnp.maximum(m, scores[t])
    es = [jnp.exp(s - m) for s in scores]
    tot = es[0]
    for t in range(1, _T):
        tot = tot + es[t]
    ae = ae_ref[...]
    ab = ab_ref[...]
    mix2 = jnp.zeros((_N, _D), jnp.float32)
    for t in range(_T):
        wt = es[t] / tot
        mt = wt * ctx_ref[t]
        bt = jnp.exp(ab * jnp.float32(-(_T - 1 - t)))
        mix2 = mix2 + jnp.maximum(ae * mt * bt, 0.0) + mt
    out_ref[...] = jnp.tanh(
        jnp.dot(mix2, woa_ref[...], preferred_element_type=jnp.float32)
        + jnp.dot(q, wob_ref[...], preferred_element_type=jnp.float32))


def _tc_mm_relu_body(p_ref, d_ref, w_ref, b_ref, out_ref):
    v = jnp.dot(p_ref[...] * d_ref[...], w_ref[...],
                preferred_element_type=jnp.float32) + b_ref[...]
    out_ref[...] = jnp.where(v > 0, v, 0.2 * v)


def _tc_tail_body(p_ref, d_ref, w_ref, b_ref, ap_ref, w1_ref, w2_ref, lwa_ref,
                  lwb_ref, lb_ref, out_ref):
    w = w_ref[...]
    b = b_ref[...]
    x2 = []
    for i in range(_NSETS):
        v = jnp.dot(p_ref[i] * d_ref[i], w,
                    preferred_element_type=jnp.float32) + b
        x2.append(jnp.where(v > 0, v, 0.2 * v))
    sub = [x2[t + 1] - x2[t] for t in range(_T - 1)]
    inv_nd = jnp.float32(1.0 / (_N * _D))
    zs = []
    for t in range(_T - 1):
        s1 = jnp.sum(sub[t], axis=0, keepdims=True)
        bm = jnp.sum(s1, axis=1, keepdims=True) * inv_nd
        apt = ap_ref[t:t + 1, :]
        u = 1.0 / (1.0 + apt * (bm - sub[t]))
        us = (u * sub[t]) / u
        z1 = jnp.sum(us, axis=0, keepdims=True)
        zs.append(jnp.sum(z1, axis=1, keepdims=True))
    v = jnp.zeros((_D, 1), jnp.float32)
    w1 = w1_ref[...]
    for t in range(_T - 1):
        v = v + w1[:, t:t + 1] * zs[t]
    a = jnp.where(v > 0, v, 0.2 * v)
    s2 = jnp.dot(w2_ref[...], a, preferred_element_type=jnp.float32)
    m = jnp.max(s2, axis=0, keepdims=True)
    e = jnp.exp(s2 - m)
    wat = e / jnp.sum(e, axis=0, keepdims=True)
    xx1 = wat[0:1, 0:1] * sub[0] + wat[2:3, 0:1] * sub[2]
    res = (jnp.dot(x2[_NSETS - 1], lwa_ref[...], preferred_element_type=jnp.float32)
           + jnp.dot(xx1, lwb_ref[...], preferred_element_type=jnp.float32)
           + lb_ref[...])
    out_ref[...] = jnp.where(res > 0, res, 0.01 * res)


def _sc_seg_body(x_hbm, idx_hbm, out_hbm, dinv_hbm, idx_row, idx_col, dinv,
                 binv, gbufs, sbuf, acc_sh, gsem, ssem):
    sid = lax.axis_index("s")
    wid = sid * 2 + lax.axis_index("c")
    nsrc = x_hbm.shape[0]
    my_acc = acc_sh.at[sid]

    @pl.when(wid < _NSETS)
    def _():
        i = wid
        isrc = lax.rem(i, nsrc)
        zeros16 = jnp.zeros((16,), jnp.float32)
        ones16 = jnp.ones((16,), jnp.float32)

        pltpu.sync_copy(idx_hbm.at[i, 0], idx_row)
        pltpu.sync_copy(idx_hbm.at[i, 1], idx_col)

        gds = [None] * _NCHUNK

        def prime(tbl, gidx):
            for b in range(4):
                gds[b] = pltpu.async_copy(tbl.at[gidx.at[b]], gbufs.at[b], gsem)

        def finish(tbl, gidx, sidx):
            sds = [None] * _NCHUNK
            ngroups = _NCHUNK // 4
            for o in range(ngroups):
                bank = (o % 2) * 4
                nbank = ((o + 1) % 2) * 4
                if o > 0:
                    for b in range(4):
                        sds[(o - 1) * 4 + b].wait()
                if o < ngroups - 1:
                    for b in range(4):
                        c = (o + 1) * 4 + b
                        gds[c] = pltpu.async_copy(tbl.at[gidx.at[c]],
                                                  gbufs.at[nbank + b], gsem)
                for b in range(4):
                    c = o * 4 + b
                    gds[c].wait()
                    sds[c] = pltpu.async_copy(gbufs.at[bank + b],
                                              my_acc.at[sidx.at[c]], ssem,
                                              add=True)
            for b in range(4):
                sds[_NCHUNK - 4 + b].wait()

        prime(x_hbm.at[isrc], idx_row)

        def zdeg(k, c):
            dinv[pl.ds(k * 16, 16)] = zeros16
            binv[pl.ds(k * 16, 16)] = zeros16
            return c

        lax.fori_loop(0, _NPAD // 16, zdeg, 0)

        def cnt(j, c):
            a = j // 8
            s = lax.rem(j, 8) * 16
            plsc.addupdate_scatter(dinv, [idx_row[a, pl.ds(s, 16)]], ones16)
            plsc.addupdate_scatter(binv, [idx_col[a, pl.ds(s, 16)]], ones16)
            return c

        lax.fori_loop(0, _NNZ // 16, cnt, 0)

        def rcp(k, c):
            sl = pl.ds(k * 16, 16)
            dv = dinv[sl]
            dinv[sl] = jnp.where(dv > 0, 1.0 / dv, 0.0)
            bv = binv[sl]
            binv[sl] = jnp.where(bv > 0, 1.0 / bv, 0.0)
            return c

        lax.fori_loop(0, _NPAD // 16, rcp, 0)
        pltpu.sync_copy(dinv, dinv_hbm.at[i])

        def zs(r, c):
            for f in range(_D // 16):
                sbuf[r, pl.ds(f * 16, 16)] = zeros16
            return c

        lax.fori_loop(0, _CHUNK, zs, 0)

        def zero_acc():
            zds = []
            for c in range(8):
                zds.append(pltpu.async_copy(
                    sbuf, my_acc.at[pl.ds(c * _CHUNK, _CHUNK)], ssem))
            zds.append(pltpu.async_copy(
                sbuf.at[pl.ds(0, 16)], my_acc.at[pl.ds(1024, 16)], ssem))
            for d in zds:
                d.wait()

        lane = lax.iota(jnp.int32, 16)

        def scale_rows(bref, sref, base, ngroups):
            def gw(k, y):
                rows = k * 16 + lane
                sv = sref[pl.ds(base + k * 16, 16)]
                for f in range(_D):
                    fv = jnp.full((16,), f, jnp.int32)
                    g = plsc.load_gather(bref, [rows, fv])
                    plsc.store_scatter(bref, [rows, fv], g * sv)
                return y

            lax.fori_loop(0, ngroups, gw, 0)

        def scale_out(sref):
            rds = [None] * 9
            wds = [None] * 9

            def issue_read(c):
                if c < 8:
                    rds[c] = pltpu.async_copy(
                        my_acc.at[pl.ds(c * _CHUNK, _CHUNK)], gbufs.at[c % 4],
                        gsem)
                else:
                    rds[c] = pltpu.async_copy(
                        my_acc.at[pl.ds(1024, 16)],
                        gbufs.at[c % 4].at[pl.ds(0, 16)], gsem)

            def issue_write(c):
                if c < 8:
                    wds[c] = pltpu.async_copy(
                        gbufs.at[c % 4], out_hbm.at[i].at[pl.ds(c * _CHUNK,
                                                               _CHUNK)], ssem)
                else:
                    wds[c] = pltpu.async_copy(
                        gbufs.at[c % 4].at[pl.ds(0, 2)],
                        out_hbm.at[i].at[pl.ds(1024, 2)], ssem)

            issue_read(0)
            issue_read(1)
            for c in range(9):
                if c >= 2:
                    wds[c - 2].wait()
                if c + 2 <= 8:
                    issue_read(c + 2)
                rds[c].wait()
                scale_rows(gbufs.at[c % 4], sref, c * _CHUNK,
                           8 if c < 8 else 1)
                issue_write(c)
            wds[7].wait()
            wds[8].wait()

        def copy_out():
            wds = []
            for c in range(8):
                wds.append(pltpu.async_copy(
                    my_acc.at[pl.ds(c * _CHUNK, _CHUNK)],
                    out_hbm.at[i].at[pl.ds(c * _CHUNK, _CHUNK)], ssem))
            wds.append(pltpu.async_copy(
                my_acc.at[pl.ds(1024, 2)], out_hbm.at[i].at[pl.ds(1024, 2)],
                ssem))
            for d in wds:
                d.wait()

        zero_acc()
        finish(x_hbm.at[isrc], idx_row, idx_col)
        scale_out(binv)
        zero_acc()
        prime(out_hbm.at[i], idx_col)
        finish(out_hbm.at[i], idx_col, idx_row)
        copy_out()


def _sc_seg(x_tables, idx_all):
    mesh = plsc.VectorSubcoreMesh(core_axis_name="c", subcore_axis_name="s",
                                  num_cores=2, num_subcores=16)
    return pl.kernel(
        _sc_seg_body,
        out_type=(jax.ShapeDtypeStruct((_NSETS, _N, _D), jnp.float32),
                  jax.ShapeDtypeStruct((_NSETS, _NPAD), jnp.float32)),
        mesh=mesh,
        compiler_params=pltpu.CompilerParams(needs_layout_passes=False,
                                             use_tc_tiling_on_sc=False),
        scratch_types=[
            pltpu.VMEM((_NCHUNK, _CHUNK), jnp.int32),
            pltpu.VMEM((_NCHUNK, _CHUNK), jnp.int32),
            pltpu.VMEM((_NPAD,), jnp.float32),
            pltpu.VMEM((_NPAD,), jnp.float32),
            pltpu.VMEM((8, _CHUNK, _D), jnp.float32),
            pltpu.VMEM((_CHUNK, _D), jnp.float32),
            pltpu.VMEM_SHARED((9, _NPAD, _D), jnp.float32),
            pltpu.SemaphoreType.DMA,
            pltpu.SemaphoreType.DMA,
        ],
    )(x_tables, idx_all)


def kernel(price_input, hyp_input_T, hyp_input, gru_Wih, gru_Whh, gru_bih,
           gru_bhh, att_Win, att_Wout, ae, ab, h1_W, h1_b, h2_W, h2_b, w1,
           w2, a_p, lin_W, lin_b):
    f32 = jnp.float32
    xs = jnp.transpose(price_input, (1, 0, 2))
    idx_all = jnp.concatenate([hyp_input_T, hyp_input[None]], axis=0)
    idx_all = idx_all.reshape(_NSETS, 2, _NCHUNK, _CHUNK)

    output = pl.pallas_call(
        _tc_gru_att_body,
        out_shape=jax.ShapeDtypeStruct((_N, _D), f32),
        scratch_shapes=[pltpu.VMEM((_T, _N, _D), f32)],
    )(xs, gru_Wih.T, gru_Whh.T, gru_bih.reshape(1, 3 * _D),
      gru_bhh.reshape(1, 3 * _D), att_Win.T, att_Wout[:, :_D].T,
      att_Wout[:, _D:].T, ae.reshape(_N, 1), ab.reshape(_N, 1))

    p1, deg1 = _sc_seg(output[None], idx_all)
    dflat = deg1[:, :_N].reshape(_NSETS * _N, 1)

    x1 = pl.pallas_call(
        _tc_mm_relu_body,
        out_shape=jax.ShapeDtypeStruct((_NSETS * _N, _D), f32),
    )(p1.reshape(_NSETS * _N, _D), dflat, h1_W.T, h1_b.reshape(1, _D))

    p2, deg2 = _sc_seg(x1.reshape(_NSETS, _N, _D), idx_all)

    out = pl.pallas_call(
        _tc_tail_body,
        out_shape=jax.ShapeDtypeStruct((_N, 1), f32),
    )(p2, deg2[:, :_N].reshape(_NSETS, _N, 1), h2_W.T, h2_b.reshape(1, _D),
      a_p.reshape(_T - 1, 1), w1, w2, lin_W[:, :_D].T, lin_W[:, _D:].T,
      lin_b.reshape(1, 1))
    return out


# speedup vs baseline: 21.5834x; 1.5100x over previous
import jax
import jax.numpy as jnp
from jax import lax
from jax.experimental import pallas as pl
from jax.experimental.pallas import tpu as pltpu
from jax.experimental.pallas import tpu_sc as plsc

_N = 1026
_T = 16
_D = 64
_NNZ = 8192
_NSETS = 17
_CHUNK = 128
_NCHUNK = _NNZ // _CHUNK
_NPAD = 1040


def _tc_gru_att_body(xs_ref, wih_ref, whh_ref, bih_ref, bhh_ref, win_ref,
                     woa_ref, wob_ref, ae_ref, ab_ref, out_ref, ctx_ref):
    wih = wih_ref[...]
    whh = whh_ref[...]
    bih = bih_ref[...]
    bhh = bhh_ref[...]
    h = jnp.zeros((_N, _D), jnp.float32)
    for t in range(_T):
        gi = jnp.dot(xs_ref[t], wih, preferred_element_type=jnp.float32) + bih
        gh = jnp.dot(h, whh, preferred_element_type=jnp.float32) + bhh
        r = jax.nn.sigmoid(gi[:, :_D] + gh[:, :_D])
        z = jax.nn.sigmoid(gi[:, _D:2 * _D] + gh[:, _D:2 * _D])
        n = jnp.tanh(gi[:, 2 * _D:] + r * gh[:, 2 * _D:])
        h = (1.0 - z) * n + z * h
        ctx_ref[t] = h
    q = jnp.dot(h, win_ref[...], preferred_element_type=jnp.float32)
    scores = [jnp.sum(q * ctx_ref[t], axis=1, keepdims=True) for t in range(_T)]
    m = scores[0]
    for t in range(1, _T):
        m = jnp.maximum(m, scores[t])
    es = [jnp.exp(s - m) for s in scores]
    tot = es[0]
    for t in range(1, _T):
        tot = tot + es[t]
    ae = ae_ref[...]
    ab = ab_ref[...]
    mix2 = jnp.zeros((_N, _D), jnp.float32)
    for t in range(_T):
        wt = es[t] / tot
        mt = wt * ctx_ref[t]
        bt = jnp.exp(ab * jnp.float32(-(_T - 1 - t)))
        mix2 = mix2 + jnp.maximum(ae * mt * bt, 0.0) + mt
    out_ref[...] = jnp.tanh(
        jnp.dot(mix2, woa_ref[...], preferred_element_type=jnp.float32)
        + jnp.dot(q, wob_ref[...], preferred_element_type=jnp.float32))


def _tc_mm_relu_body(p_ref, d_ref, w_ref, b_ref, out_ref):
    v = jnp.dot(p_ref[...] * d_ref[...], w_ref[...],
                preferred_element_type=jnp.float32) + b_ref[...]
    out_ref[...] = jnp.where(v > 0, v, 0.2 * v)


def _tc_tail_body(p_ref, d_ref, w_ref, b_ref, ap_ref, w1_ref, w2_ref, lwa_ref,
                  lwb_ref, lb_ref, out_ref):
    w = w_ref[...]
    b = b_ref[...]
    x2 = []
    for i in range(_NSETS):
        v = jnp.dot(p_ref[i] * d_ref[i], w,
                    preferred_element_type=jnp.float32) + b
        x2.append(jnp.where(v > 0, v, 0.2 * v))
    sub = [x2[t + 1] - x2[t] for t in range(_T - 1)]
    inv_nd = jnp.float32(1.0 / (_N * _D))
    zs = []
    for t in range(_T - 1):
        s1 = jnp.sum(sub[t], axis=0, keepdims=True)
        bm = jnp.sum(s1, axis=1, keepdims=True) * inv_nd
        apt = ap_ref[t:t + 1, :]
        u = 1.0 / (1.0 + apt * (bm - sub[t]))
        us = (u * sub[t]) / u
        z1 = jnp.sum(us, axis=0, keepdims=True)
        zs.append(jnp.sum(z1, axis=1, keepdims=True))
    v = jnp.zeros((_D, 1), jnp.float32)
    w1 = w1_ref[...]
    for t in range(_T - 1):
        v = v + w1[:, t:t + 1] * zs[t]
    a = jnp.where(v > 0, v, 0.2 * v)
    s2 = jnp.dot(w2_ref[...], a, preferred_element_type=jnp.float32)
    m = jnp.max(s2, axis=0, keepdims=True)
    e = jnp.exp(s2 - m)
    wat = e / jnp.sum(e, axis=0, keepdims=True)
    xx1 = wat[0:1, 0:1] * sub[0] + wat[2:3, 0:1] * sub[2]
    res = (jnp.dot(x2[_NSETS - 1], lwa_ref[...], preferred_element_type=jnp.float32)
           + jnp.dot(xx1, lwb_ref[...], preferred_element_type=jnp.float32)
           + lb_ref[...])
    out_ref[...] = jnp.where(res > 0, res, 0.01 * res)


def _sc_seg_body(x_hbm, idx_hbm, out_hbm, dinv_hbm, idx_row, idx_col, dinv,
                 binv, gbufs, sbuf, acc_sh, gsem, ssem):
    sid = lax.axis_index("s")
    wid = sid * 2 + lax.axis_index("c")
    nsrc = x_hbm.shape[0]
    my_acc = acc_sh.at[sid]

    @pl.when(wid < _NSETS)
    def _():
        i = wid
        isrc = lax.rem(i, nsrc)
        zeros16 = jnp.zeros((16,), jnp.float32)
        ones16 = jnp.ones((16,), jnp.float32)

        pltpu.sync_copy(idx_hbm.at[i, 0], idx_row)
        pltpu.sync_copy(idx_hbm.at[i, 1], idx_col)

        gds = [None] * _NCHUNK

        def prime(tbl, gidx):
            for b in range(4):
                gds[b] = pltpu.async_copy(tbl.at[gidx.at[b]], gbufs.at[b], gsem)

        def finish(tbl, gidx, sidx):
            sds = [None] * _NCHUNK
            ngroups = _NCHUNK // 4
            for o in range(ngroups):
                bank = (o % 2) * 4
                nbank = ((o + 1) % 2) * 4
                if o > 0:
                    for b in range(4):
                        sds[(o - 1) * 4 + b].wait()
                if o < ngroups - 1:
                    for b in range(4):
                        c = (o + 1) * 4 + b
                        gds[c] = pltpu.async_copy(tbl.at[gidx.at[c]],
                                                  gbufs.at[nbank + b], gsem)
                for b in range(4):
                    c = o * 4 + b
                    gds[c].wait()
                    sds[c] = pltpu.async_copy(gbufs.at[bank + b],
                                              my_acc.at[sidx.at[c]], ssem,
                                              add=True)
            for b in range(4):
                sds[_NCHUNK - 4 + b].wait()

        prime(x_hbm.at[isrc], idx_row)

        def zdeg(k, c):
            dinv[pl.ds(k * 16, 16)] = zeros16
            binv[pl.ds(k * 16, 16)] = zeros16
            return c

        lax.fori_loop(0, _NPAD // 16, zdeg, 0)

        def cnt(j, c):
            a = j // 8
            s = lax.rem(j, 8) * 16
            plsc.addupdate_scatter(dinv, [idx_row[a, pl.ds(s, 16)]], ones16)
            plsc.addupdate_scatter(binv, [idx_col[a, pl.ds(s, 16)]], ones16)
            return c

        lax.fori_loop(0, _NNZ // 16, cnt, 0)

        def rcp(k, c):
            sl = pl.ds(k * 16, 16)
            dv = dinv[sl]
            dinv[sl] = jnp.where(dv > 0, 1.0 / dv, 0.0)
            bv = binv[sl]
            binv[sl] = jnp.where(bv > 0, 1.0 / bv, 0.0)
            return c

        lax.fori_loop(0, _NPAD // 16, rcp, 0)
        pltpu.sync_copy(dinv, dinv_hbm.at[i])

        def zs(r, c):
            for f in range(_D // 16):
                sbuf[r, pl.ds(f * 16, 16)] = zeros16
            return c

        lax.fori_loop(0, _CHUNK, zs, 0)

        def zero_acc():
            zds = []
            for c in range(8):
                zds.append(pltpu.async_copy(
                    sbuf, my_acc.at[pl.ds(c * _CHUNK, _CHUNK)], ssem))
            zds.append(pltpu.async_copy(
                sbuf.at[pl.ds(0, 16)], my_acc.at[pl.ds(1024, 16)], ssem))
            for d in zds:
                d.wait()

        lane = lax.iota(jnp.int32, 16)

        def scale_rows(bref, sref, base, ngroups):
            def gw(k, y):
                sv = sref[pl.ds(base + k * 16, 16)]
                for r in range(16):
                    bc = jnp.full((16,), sv[r], jnp.float32)
                    row = k * 16 + r
                    for f in range(_D // 16):
                        sl = pl.ds(f * 16, 16)
                        bref[row, sl] = bref[row, sl] * bc
                return y

            lax.fori_loop(0, ngroups, gw, 0)

        def scale_out(sref):
            rds = [None] * 9
            wds = [None] * 9

            def issue_read(c):
                if c < 8:
                    rds[c] = pltpu.async_copy(
                        my_acc.at[pl.ds(c * _CHUNK, _CHUNK)], gbufs.at[c % 4],
                        gsem)
                else:
                    rds[c] = pltpu.async_copy(
                        my_acc.at[pl.ds(1024, 16)],
                        gbufs.at[c % 4].at[pl.ds(0, 16)], gsem)

            def issue_write(c):
                if c < 8:
                    wds[c] = pltpu.async_copy(
                        gbufs.at[c % 4], out_hbm.at[i].at[pl.ds(c * _CHUNK,
                                                               _CHUNK)], ssem)
                else:
                    wds[c] = pltpu.async_copy(
                        gbufs.at[c % 4].at[pl.ds(0, 2)],
                        out_hbm.at[i].at[pl.ds(1024, 2)], ssem)

            issue_read(0)
            issue_read(1)
            for c in range(9):
                if c >= 2:
                    wds[c - 2].wait()
                if c + 2 <= 8:
                    issue_read(c + 2)
                rds[c].wait()
                scale_rows(gbufs.at[c % 4], sref, c * _CHUNK,
                           8 if c < 8 else 1)
                issue_write(c)
            wds[7].wait()
            wds[8].wait()

        def copy_out():
            wds = []
            for c in range(8):
                wds.append(pltpu.async_copy(
                    my_acc.at[pl.ds(c * _CHUNK, _CHUNK)],
                    out_hbm.at[i].at[pl.ds(c * _CHUNK, _CHUNK)], ssem))
            wds.append(pltpu.async_copy(
                my_acc.at[pl.ds(1024, 2)], out_hbm.at[i].at[pl.ds(1024, 2)],
                ssem))
            for d in wds:
                d.wait()

        zero_acc()
        finish(x_hbm.at[isrc], idx_row, idx_col)
        scale_out(binv)
        zero_acc()
        prime(out_hbm.at[i], idx_col)
        finish(out_hbm.at[i], idx_col, idx_row)
        copy_out()


def _sc_seg(x_tables, idx_all):
    mesh = plsc.VectorSubcoreMesh(core_axis_name="c", subcore_axis_name="s",
                                  num_cores=2, num_subcores=16)
    return pl.kernel(
        _sc_seg_body,
        out_type=(jax.ShapeDtypeStruct((_NSETS, _N, _D), jnp.float32),
                  jax.ShapeDtypeStruct((_NSETS, _NPAD), jnp.float32)),
        mesh=mesh,
        compiler_params=pltpu.CompilerParams(needs_layout_passes=False,
                                             use_tc_tiling_on_sc=False),
        scratch_types=[
            pltpu.VMEM((_NCHUNK, _CHUNK), jnp.int32),
            pltpu.VMEM((_NCHUNK, _CHUNK), jnp.int32),
            pltpu.VMEM((_NPAD,), jnp.float32),
            pltpu.VMEM((_NPAD,), jnp.float32),
            pltpu.VMEM((8, _CHUNK, _D), jnp.float32),
            pltpu.VMEM((_CHUNK, _D), jnp.float32),
            pltpu.VMEM_SHARED((9, _NPAD, _D), jnp.float32),
            pltpu.SemaphoreType.DMA,
            pltpu.SemaphoreType.DMA,
        ],
    )(x_tables, idx_all)


def kernel(price_input, hyp_input_T, hyp_input, gru_Wih, gru_Whh, gru_bih,
           gru_bhh, att_Win, att_Wout, ae, ab, h1_W, h1_b, h2_W, h2_b, w1,
           w2, a_p, lin_W, lin_b):
    f32 = jnp.float32
    xs = jnp.transpose(price_input, (1, 0, 2))
    idx_all = jnp.concatenate([hyp_input_T, hyp_input[None]], axis=0)
    idx_all = idx_all.reshape(_NSETS, 2, _NCHUNK, _CHUNK)

    output = pl.pallas_call(
        _tc_gru_att_body,
        out_shape=jax.ShapeDtypeStruct((_N, _D), f32),
        scratch_shapes=[pltpu.VMEM((_T, _N, _D), f32)],
    )(xs, gru_Wih.T, gru_Whh.T, gru_bih.reshape(1, 3 * _D),
      gru_bhh.reshape(1, 3 * _D), att_Win.T, att_Wout[:, :_D].T,
      att_Wout[:, _D:].T, ae.reshape(_N, 1), ab.reshape(_N, 1))

    p1, deg1 = _sc_seg(output[None], idx_all)
    dflat = deg1[:, :_N].reshape(_NSETS * _N, 1)

    x1 = pl.pallas_call(
        _tc_mm_relu_body,
        out_shape=jax.ShapeDtypeStruct((_NSETS * _N, _D), f32),
    )(p1.reshape(_NSETS * _N, _D), dflat, h1_W.T, h1_b.reshape(1, _D))

    p2, deg2 = _sc_seg(x1.reshape(_NSETS, _N, _D), idx_all)

    out = pl.pallas_call(
        _tc_tail_body,
        out_shape=jax.ShapeDtypeStruct((_N, 1), f32),
    )(p2, deg2[:, :_N].reshape(_NSETS, _N, 1), h2_W.T, h2_b.reshape(1, _D),
      a_p.reshape(_T - 1, 1), w1, w2, lin_W[:, :_D].T, lin_W[:, _D:].T,
      lin_b.reshape(1, 1))
    return out


# speedup vs baseline: 22.9334x; 1.0625x over previous
import jax
import jax.numpy as jnp
from jax import lax
from jax.experimental import pallas as pl
from jax.experimental.pallas import tpu as pltpu
from jax.experimental.pallas import tpu_sc as plsc

_N = 1026
_T = 16
_D = 64
_NNZ = 8192
_NSETS = 17
_CHUNK = 128
_NCHUNK = _NNZ // _CHUNK
_NPAD = 1040


def _tc_gru_att_body(xs_ref, wih_ref, whh_ref, bih_ref, bhh_ref, win_ref,
                     woa_ref, wob_ref, ae_ref, ab_ref, out_ref, ctx_ref):
    wih = wih_ref[...]
    whh = whh_ref[...]
    bih = bih_ref[...]
    bhh = bhh_ref[...]
    h = jnp.zeros((_N, _D), jnp.float32)
    for t in range(_T):
        gi = jnp.dot(xs_ref[t], wih, preferred_element_type=jnp.float32) + bih
        gh = jnp.dot(h, whh, preferred_element_type=jnp.float32) + bhh
        r = jax.nn.sigmoid(gi[:, :_D] + gh[:, :_D])
        z = jax.nn.sigmoid(gi[:, _D:2 * _D] + gh[:, _D:2 * _D])
        n = jnp.tanh(gi[:, 2 * _D:] + r * gh[:, 2 * _D:])
        h = (1.0 - z) * n + z * h
        ctx_ref[t] = h
    q = jnp.dot(h, win_ref[...], preferred_element_type=jnp.float32)
    scores = [jnp.sum(q * ctx_ref[t], axis=1, keepdims=True) for t in range(_T)]
    m = scores[0]
    for t in range(1, _T):
        m = jnp.maximum(m, scores[t])
    es = [jnp.exp(s - m) for s in scores]
    tot = es[0]
    for t in range(1, _T):
        tot = tot + es[t]
    ae = ae_ref[...]
    ab = ab_ref[...]
    mix2 = jnp.zeros((_N, _D), jnp.float32)
    for t in range(_T):
        wt = es[t] / tot
        mt = wt * ctx_ref[t]
        bt = jnp.exp(ab * jnp.float32(-(_T - 1 - t)))
        mix2 = mix2 + jnp.maximum(ae * mt * bt, 0.0) + mt
    out_ref[...] = jnp.tanh(
        jnp.dot(mix2, woa_ref[...], preferred_element_type=jnp.float32)
        + jnp.dot(q, wob_ref[...], preferred_element_type=jnp.float32))


def _tc_mm_relu_body(p_ref, d_ref, w_ref, b_ref, out_ref):
    v = jnp.dot(p_ref[...] * d_ref[...], w_ref[...],
                preferred_element_type=jnp.float32) + b_ref[...]
    out_ref[...] = jnp.where(v > 0, v, 0.2 * v)


def _tc_tail_body(p_ref, d_ref, w_ref, b_ref, ap_ref, w1_ref, w2_ref, lwa_ref,
                  lwb_ref, lb_ref, out_ref):
    w = w_ref[...]
    b = b_ref[...]
    x2 = []
    for i in range(_NSETS):
        v = jnp.dot(p_ref[i] * d_ref[i], w,
                    preferred_element_type=jnp.float32) + b
        x2.append(jnp.where(v > 0, v, 0.2 * v))
    sub = [x2[t + 1] - x2[t] for t in range(_T - 1)]
    inv_nd = jnp.float32(1.0 / (_N * _D))
    zs = []
    for t in range(_T - 1):
        s1 = jnp.sum(sub[t], axis=0, keepdims=True)
        bm = jnp.sum(s1, axis=1, keepdims=True) * inv_nd
        apt = ap_ref[t:t + 1, :]
        u = 1.0 / (1.0 + apt * (bm - sub[t]))
        us = (u * sub[t]) / u
        z1 = jnp.sum(us, axis=0, keepdims=True)
        zs.append(jnp.sum(z1, axis=1, keepdims=True))
    v = jnp.zeros((_D, 1), jnp.float32)
    w1 = w1_ref[...]
    for t in range(_T - 1):
        v = v + w1[:, t:t + 1] * zs[t]
    a = jnp.where(v > 0, v, 0.2 * v)
    s2 = jnp.dot(w2_ref[...], a, preferred_element_type=jnp.float32)
    m = jnp.max(s2, axis=0, keepdims=True)
    e = jnp.exp(s2 - m)
    wat = e / jnp.sum(e, axis=0, keepdims=True)
    xx1 = wat[0:1, 0:1] * sub[0] + wat[2:3, 0:1] * sub[2]
    res = (jnp.dot(x2[_NSETS - 1], lwa_ref[...], preferred_element_type=jnp.float32)
           + jnp.dot(xx1, lwb_ref[...], preferred_element_type=jnp.float32)
           + lb_ref[...])
    out_ref[...] = jnp.where(res > 0, res, 0.01 * res)


def _sc_seg_body(x_hbm, idx_hbm, out_hbm, dinv_hbm, idx_row, idx_col, dinv,
                 binv, gbufs, sbuf, acc_sh, gsem, ssem):
    sid = lax.axis_index("s")
    wid = sid * 2 + lax.axis_index("c")
    nsrc = x_hbm.shape[0]
    my_acc = acc_sh.at[sid]

    @pl.when(wid < _NSETS)
    def _():
        i = wid
        isrc = lax.rem(i, nsrc)
        zeros16 = jnp.zeros((16,), jnp.float32)
        ones16 = jnp.ones((16,), jnp.float32)

        pltpu.sync_copy(idx_hbm.at[i, 0], idx_row)
        pltpu.sync_copy(idx_hbm.at[i, 1], idx_col)

        gds = [None] * _NCHUNK

        def prime(tbl, gidx):
            for b in range(4):
                gds[b] = pltpu.async_copy(tbl.at[gidx.at[b]], gbufs.at[b], gsem)

        def finish(tbl, gidx, sidx, work=None):
            sds = [None] * _NCHUNK
            ngroups = _NCHUNK // 4
            for o in range(ngroups):
                bank = (o % 2) * 4
                nbank = ((o + 1) % 2) * 4
                if o > 0:
                    for b in range(4):
                        sds[(o - 1) * 4 + b].wait()
                if o < ngroups - 1:
                    for b in range(4):
                        c = (o + 1) * 4 + b
                        gds[c] = pltpu.async_copy(tbl.at[gidx.at[c]],
                                                  gbufs.at[nbank + b], gsem)
                if work is not None:
                    work(o)
                for b in range(4):
                    c = o * 4 + b
                    gds[c].wait()
                    sds[c] = pltpu.async_copy(gbufs.at[bank + b],
                                              my_acc.at[sidx.at[c]], ssem,
                                              add=True)
            for b in range(4):
                sds[_NCHUNK - 4 + b].wait()

        prime(x_hbm.at[isrc], idx_row)

        def zdeg(k, c):
            dinv[pl.ds(k * 16, 16)] = zeros16
            binv[pl.ds(k * 16, 16)] = zeros16
            return c

        lax.fori_loop(0, _NPAD // 16, zdeg, 0)

        def cnt(j, c):
            a = j // 8
            s = lax.rem(j, 8) * 16
            plsc.addupdate_scatter(dinv, [idx_row[a, pl.ds(s, 16)]], ones16)
            plsc.addupdate_scatter(binv, [idx_col[a, pl.ds(s, 16)]], ones16)
            return c

        def cnt_slice(o):
            lax.fori_loop(o * 32, (o + 1) * 32, cnt, 0)

        def rcp(k, c):
            sl = pl.ds(k * 16, 16)
            dv = dinv[sl]
            dinv[sl] = jnp.where(dv > 0, 1.0 / dv, 0.0)
            bv = binv[sl]
            binv[sl] = jnp.where(bv > 0, 1.0 / bv, 0.0)
            return c

        def zs(r, c):
            for f in range(_D // 16):
                sbuf[r, pl.ds(f * 16, 16)] = zeros16
            return c

        lax.fori_loop(0, _CHUNK, zs, 0)

        def zero_acc():
            zds = []
            for c in range(8):
                zds.append(pltpu.async_copy(
                    sbuf, my_acc.at[pl.ds(c * _CHUNK, _CHUNK)], ssem))
            zds.append(pltpu.async_copy(
                sbuf.at[pl.ds(0, 16)], my_acc.at[pl.ds(1024, 16)], ssem))
            for d in zds:
                d.wait()

        lane = lax.iota(jnp.int32, 16)

        def scale_rows(bref, sref, base, ngroups):
            def gw(k, y):
                sv = sref[pl.ds(base + k * 16, 16)]
                for r in range(16):
                    bc = jnp.full((16,), sv[r], jnp.float32)
                    row = k * 16 + r
                    for f in range(_D // 16):
                        sl = pl.ds(f * 16, 16)
                        bref[row, sl] = bref[row, sl] * bc
                return y

            lax.fori_loop(0, ngroups, gw, 0)

        def scale_out(sref):
            rds = [None] * 9
            wds = [None] * 9

            def issue_read(c):
                if c < 8:
                    rds[c] = pltpu.async_copy(
                        my_acc.at[pl.ds(c * _CHUNK, _CHUNK)], gbufs.at[c % 4],
                        gsem)
                else:
                    rds[c] = pltpu.async_copy(
                        my_acc.at[pl.ds(1024, 16)],
                        gbufs.at[c % 4].at[pl.ds(0, 16)], gsem)

            def issue_write(c):
                if c < 8:
                    wds[c] = pltpu.async_copy(
                        gbufs.at[c % 4], out_hbm.at[i].at[pl.ds(c * _CHUNK,
                                                               _CHUNK)], ssem)
                else:
                    wds[c] = pltpu.async_copy(
                        gbufs.at[c % 4].at[pl.ds(0, 2)],
                        out_hbm.at[i].at[pl.ds(1024, 2)], ssem)

            issue_read(0)
            issue_read(1)
            for c in range(9):
                if c >= 2:
                    wds[c - 2].wait()
                if c + 2 <= 8:
                    issue_read(c + 2)
                rds[c].wait()
                scale_rows(gbufs.at[c % 4], sref, c * _CHUNK,
                           8 if c < 8 else 1)
                issue_write(c)
            wds[7].wait()
            wds[8].wait()

        def copy_out():
            wds = []
            for c in range(8):
                wds.append(pltpu.async_copy(
                    my_acc.at[pl.ds(c * _CHUNK, _CHUNK)],
                    out_hbm.at[i].at[pl.ds(c * _CHUNK, _CHUNK)], ssem))
            wds.append(pltpu.async_copy(
                my_acc.at[pl.ds(1024, 2)], out_hbm.at[i].at[pl.ds(1024, 2)],
                ssem))
            for d in wds:
                d.wait()

        zero_acc()
        finish(x_hbm.at[isrc], idx_row, idx_col, work=cnt_slice)
        lax.fori_loop(0, _NPAD // 16, rcp, 0)
        pltpu.sync_copy(dinv, dinv_hbm.at[i])
        scale_out(binv)
        prime(out_hbm.at[i], idx_col)
        zero_acc()
        finish(out_hbm.at[i], idx_col, idx_row)
        copy_out()


def _sc_seg(x_tables, idx_all):
    mesh = plsc.VectorSubcoreMesh(core_axis_name="c", subcore_axis_name="s",
                                  num_cores=2, num_subcores=16)
    return pl.kernel(
        _sc_seg_body,
        out_type=(jax.ShapeDtypeStruct((_NSETS, _N, _D), jnp.float32),
                  jax.ShapeDtypeStruct((_NSETS, _NPAD), jnp.float32)),
        mesh=mesh,
        compiler_params=pltpu.CompilerParams(needs_layout_passes=False,
                                             use_tc_tiling_on_sc=False),
        scratch_types=[
            pltpu.VMEM((_NCHUNK, _CHUNK), jnp.int32),
            pltpu.VMEM((_NCHUNK, _CHUNK), jnp.int32),
            pltpu.VMEM((_NPAD,), jnp.float32),
            pltpu.VMEM((_NPAD,), jnp.float32),
            pltpu.VMEM((8, _CHUNK, _D), jnp.float32),
            pltpu.VMEM((_CHUNK, _D), jnp.float32),
            pltpu.VMEM_SHARED((9, _NPAD, _D), jnp.float32),
            pltpu.SemaphoreType.DMA,
            pltpu.SemaphoreType.DMA,
        ],
    )(x_tables, idx_all)


def kernel(price_input, hyp_input_T, hyp_input, gru_Wih, gru_Whh, gru_bih,
           gru_bhh, att_Win, att_Wout, ae, ab, h1_W, h1_b, h2_W, h2_b, w1,
           w2, a_p, lin_W, lin_b):
    f32 = jnp.float32
    xs = jnp.transpose(price_input, (1, 0, 2))
    idx_all = jnp.concatenate([hyp_input_T, hyp_input[None]], axis=0)
    idx_all = idx_all.reshape(_NSETS, 2, _NCHUNK, _CHUNK)

    output = pl.pallas_call(
        _tc_gru_att_body,
        out_shape=jax.ShapeDtypeStruct((_N, _D), f32),
        scratch_shapes=[pltpu.VMEM((_T, _N, _D), f32)],
    )(xs, gru_Wih.T, gru_Whh.T, gru_bih.reshape(1, 3 * _D),
      gru_bhh.reshape(1, 3 * _D), att_Win.T, att_Wout[:, :_D].T,
      att_Wout[:, _D:].T, ae.reshape(_N, 1), ab.reshape(_N, 1))

    p1, deg1 = _sc_seg(output[None], idx_all)
    dflat = deg1[:, :_N].reshape(_NSETS * _N, 1)

    x1 = pl.pallas_call(
        _tc_mm_relu_body,
        out_shape=jax.ShapeDtypeStruct((_NSETS * _N, _D), f32),
    )(p1.reshape(_NSETS * _N, _D), dflat, h1_W.T, h1_b.reshape(1, _D))

    p2, deg2 = _sc_seg(x1.reshape(_NSETS, _N, _D), idx_all)

    out = pl.pallas_call(
        _tc_tail_body,
        out_shape=jax.ShapeDtypeStruct((_N, 1), f32),
    )(p2, deg2[:, :_N].reshape(_NSETS, _N, 1), h2_W.T, h2_b.reshape(1, _D),
      a_p.reshape(_T - 1, 1), w1, w2, lin_W[:, :_D].T, lin_W[:, _D:].T,
      lin_b.reshape(1, 1))
    return out
